# Initial kernel scaffold; baseline (speedup 1.0000x reference)
import jax
import jax.numpy as jnp
from jax.experimental import pallas as pl


def kernel(x, edge_index, edge_attr, W, att_src, att_dst, bias):
    raise NotImplementedError("write your pallas kernel here")


# speedup vs baseline: 82.2679x; 82.2679x over previous
import functools

import jax
import jax.numpy as jnp
from jax import lax
from jax.experimental import pallas as pl
from jax.experimental.pallas import tpu as pltpu
from jax.experimental.pallas import tpu_sc as plsc

N = 10000
E = 320000
D_IN = 192
C_OUT = 2

NC = 2
NS = 16
L = 16
NW = NC * NS

E_TOT = E + N
EPW = ((E_TOT + NW - 1) // NW + 15) // 16 * 16
EP = EPW * NW
NIT = EPW // L


def _prologue_body(x_ref, w_ref, as_ref, ad_ref, h_ref, a2_ref, sh_ref):
    h = jnp.dot(x_ref[...], w_ref[...], preferred_element_type=jnp.float32)
    h_ref[...] = h
    asv = jnp.sum(h * as_ref[...], axis=1, keepdims=True)
    adv = jnp.sum(h * ad_ref[...], axis=1, keepdims=True)
    a2_ref[...] = jnp.concatenate([asv, adv], axis=1)
    m = jnp.max(asv) + jnp.max(adv)
    sh = jnp.where(m >= 0.0, m, 0.2 * m)
    sh_ref[...] = jnp.full((L,), sh, jnp.float32)


def _sc_body(a2_hbm, h_hbm, sh_hbm, src_hbm, dst_hbm, z_hbm,
             dp_hbm, n0p_hbm, n1p_hbm,
             a2_v, h_v, sh_v, src_v, dst_v, den_v, n0_v, n1_v):
    wid = lax.axis_index("s") * NC + lax.axis_index("c")
    pltpu.sync_copy(a2_hbm, a2_v)
    pltpu.sync_copy(h_hbm, h_v)
    pltpu.sync_copy(sh_hbm, sh_v)
    base = pl.multiple_of(wid * EPW, 8)
    pltpu.sync_copy(src_hbm.at[pl.ds(base, EPW)], src_v)
    pltpu.sync_copy(dst_hbm.at[pl.ds(base, EPW)], dst_v)
    pltpu.sync_copy(z_hbm, den_v)
    pltpu.sync_copy(z_hbm, n0_v)
    pltpu.sync_copy(z_hbm, n1_v)

    shift = sh_v[...]
    one = jnp.ones((L,), jnp.int32)
    lane = lax.iota(jnp.int32, L)

    def step(i, carry):
        off = pl.multiple_of(i * L, 8)
        s = src_v[pl.ds(off, L)]
        d = dst_v[pl.ds(off, L)]
        s2 = s + s
        d2 = d + d
        av = plsc.load_gather(a2_v, [s2]) + plsc.load_gather(a2_v, [d2 + one])
        av = jnp.where(av >= 0.0, av, 0.2 * av) - shift
        ex = jnp.exp(av)
        h0 = plsc.load_gather(h_v, [s2])
        h1 = plsc.load_gather(h_v, [s2 + one])
        m = (base + off + lane) < E_TOT
        plsc.addupdate_scatter(den_v, [d], ex, mask=m)
        plsc.addupdate_scatter(n0_v, [d], ex * h0, mask=m)
        plsc.addupdate_scatter(n1_v, [d], ex * h1, mask=m)
        return carry

    lax.fori_loop(0, NIT, step, 0)

    pltpu.sync_copy(den_v, dp_hbm.at[wid])
    pltpu.sync_copy(n0_v, n0p_hbm.at[wid])
    pltpu.sync_copy(n1_v, n1p_hbm.at[wid])


def _epilogue_body(dp_ref, n0_ref, n1_ref, b_ref, out_ref):
    den = jnp.sum(dp_ref[...], axis=0, keepdims=True)
    n0 = jnp.sum(n0_ref[...], axis=0, keepdims=True)
    n1 = jnp.sum(n1_ref[...], axis=0, keepdims=True)
    inv = 1.0 / (den + 1e-16)
    out_ref[...] = jnp.concatenate([n0 * inv, n1 * inv], axis=0) + b_ref[...]


@jax.jit
def kernel(x, edge_index, edge_attr, W, att_src, att_dst, bias):
    del edge_attr
    f32 = jnp.float32

    prologue = pl.pallas_call(
        _prologue_body,
        out_shape=[
            jax.ShapeDtypeStruct((N, C_OUT), f32),
            jax.ShapeDtypeStruct((N, 2), f32),
            jax.ShapeDtypeStruct((L,), f32),
        ],
    )
    h, a2, sh = prologue(x, W, att_src.reshape(1, C_OUT), att_dst.reshape(1, C_OUT))

    loop = jnp.arange(N, dtype=jnp.int32)
    pad = jnp.zeros((EP - E_TOT,), dtype=jnp.int32)
    src_p = jnp.concatenate([edge_index[0], loop, pad])
    dst_p = jnp.concatenate([edge_index[1], loop, pad])
    zeros_n = jnp.zeros((N,), f32)

    mesh = plsc.VectorSubcoreMesh(
        core_axis_name="c", subcore_axis_name="s", num_cores=NC, num_subcores=NS)
    sc = pl.kernel(
        _sc_body,
        out_type=(
            jax.ShapeDtypeStruct((NW, N), f32),
            jax.ShapeDtypeStruct((NW, N), f32),
            jax.ShapeDtypeStruct((NW, N), f32),
        ),
        mesh=mesh,
        compiler_params=pltpu.CompilerParams(
            needs_layout_passes=False, use_tc_tiling_on_sc=False),
        scratch_types=[
            pltpu.VMEM((2 * N,), f32),
            pltpu.VMEM((2 * N,), f32),
            pltpu.VMEM((L,), f32),
            pltpu.VMEM((EPW,), jnp.int32),
            pltpu.VMEM((EPW,), jnp.int32),
            pltpu.VMEM((N,), f32),
            pltpu.VMEM((N,), f32),
            pltpu.VMEM((N,), f32),
        ],
    )
    dp, n0p, n1p = sc(a2.reshape(2 * N), h.reshape(2 * N), sh, src_p, dst_p, zeros_n)

    epilogue = pl.pallas_call(
        _epilogue_body,
        out_shape=jax.ShapeDtypeStruct((2, N), f32),
    )
    out2 = epilogue(dp, n0p, n1p, bias.reshape(C_OUT, 1))
    return out2.T


# speedup vs baseline: 98.9451x; 1.2027x over previous
import jax
import jax.numpy as jnp
from jax import lax
from jax.experimental import pallas as pl
from jax.experimental.pallas import tpu as pltpu
from jax.experimental.pallas import tpu_sc as plsc

N = 10000
E = 320000
D_IN = 192
C_OUT = 2

NC = 2
NS = 16
L = 16
NW = NC * NS

U = 5
EPW = E // NW
NIT = EPW // (U * L)
ZIT = N // L
NSL = ((N + NW - 1) // NW + L - 1) // L * L


def _prologue_body(x_ref, w_ref, as_ref, ad_ref,
                   asv_ref, adv_ref, h0_ref, h1_ref, sh_ref):
    h = jnp.dot(x_ref[...], w_ref[...], preferred_element_type=jnp.float32)
    asv = jnp.sum(h * as_ref[...], axis=1, keepdims=True)
    adv = jnp.sum(h * ad_ref[...], axis=1, keepdims=True)
    asv_ref[...] = asv
    adv_ref[...] = adv
    h0_ref[...] = h[:, 0:1]
    h1_ref[...] = h[:, 1:2]
    m = jnp.max(asv) + jnp.max(adv)
    sh = jnp.where(m >= 0.0, m, 0.2 * m)
    sh_ref[...] = jnp.full((L,), sh, jnp.float32)


def _sc_body(as_hbm, ad_hbm, h0_hbm, h1_hbm, sh_hbm, ei_hbm,
             dp_hbm, n0p_hbm, n1p_hbm,
             as_v, ad_v, h0_v, h1_v, sh_v, src_v, dst_v,
             den_v, n0_v, n1_v, sem):
    wid = lax.axis_index("s") * NC + lax.axis_index("c")
    base = pl.multiple_of(wid * EPW, 8)
    copies = [
        pltpu.async_copy(as_hbm, as_v, sem),
        pltpu.async_copy(ad_hbm, ad_v, sem),
        pltpu.async_copy(h0_hbm, h0_v, sem),
        pltpu.async_copy(h1_hbm, h1_v, sem),
        pltpu.async_copy(sh_hbm, sh_v, sem),
        pltpu.async_copy(ei_hbm.at[0, pl.ds(base, EPW)], src_v, sem),
        pltpu.async_copy(ei_hbm.at[1, pl.ds(base, EPW)], dst_v, sem),
    ]

    z = jnp.zeros((L,), jnp.float32)

    def zero_step(i, carry):
        off = pl.multiple_of(i * L, 8)
        den_v[pl.ds(off, L)] = z
        n0_v[pl.ds(off, L)] = z
        n1_v[pl.ds(off, L)] = z
        return carry

    lax.fori_loop(0, ZIT, zero_step, 0)
    for c in copies:
        c.wait()

    shift = sh_v[...]

    def step(i, carry):
        block = pl.multiple_of(i * (U * L), 8)
        for u in range(U):
            off = block + u * L
            s = src_v[pl.ds(off, L)]
            d = dst_v[pl.ds(off, L)]
            av = plsc.load_gather(as_v, [s]) + plsc.load_gather(ad_v, [d])
            av = jnp.maximum(av, 0.2 * av) - shift
            ex = jnp.exp(av)
            h0 = plsc.load_gather(h0_v, [s])
            h1 = plsc.load_gather(h1_v, [s])
            plsc.addupdate_scatter(den_v, [d], ex)
            plsc.addupdate_scatter(n0_v, [d], ex * h0)
            plsc.addupdate_scatter(n1_v, [d], ex * h1)
        return carry

    lax.fori_loop(0, NIT, step, 0)

    nbase = pl.multiple_of(wid * NSL, 8)
    nits = (jnp.minimum(NSL, N - nbase) + L - 1) // L

    def loop_step(j, carry):
        off = nbase + j * L
        sl = pl.ds(off, L)
        av = as_v[sl] + ad_v[sl]
        av = jnp.maximum(av, 0.2 * av) - shift
        ex = jnp.exp(av)
        plsc.addupdate(den_v.at[sl], ex)
        plsc.addupdate(n0_v.at[sl], ex * h0_v[sl])
        plsc.addupdate(n1_v.at[sl], ex * h1_v[sl])
        return carry

    lax.fori_loop(0, nits, loop_step, 0)

    pltpu.sync_copy(den_v, dp_hbm.at[wid])
    pltpu.sync_copy(n0_v, n0p_hbm.at[wid])
    pltpu.sync_copy(n1_v, n1p_hbm.at[wid])


def _epilogue_body(dp_ref, n0_ref, n1_ref, b_ref, out_ref):
    den = jnp.sum(dp_ref[...], axis=0, keepdims=True)
    n0 = jnp.sum(n0_ref[...], axis=0, keepdims=True)
    n1 = jnp.sum(n1_ref[...], axis=0, keepdims=True)
    inv = 1.0 / (den + 1e-16)
    out_ref[...] = jnp.concatenate([n0 * inv, n1 * inv], axis=0) + b_ref[...]


@jax.jit
def kernel(x, edge_index, edge_attr, W, att_src, att_dst, bias):
    del edge_attr
    f32 = jnp.float32

    prologue = pl.pallas_call(
        _prologue_body,
        out_shape=[
            jax.ShapeDtypeStruct((N, 1), f32),
            jax.ShapeDtypeStruct((N, 1), f32),
            jax.ShapeDtypeStruct((N, 1), f32),
            jax.ShapeDtypeStruct((N, 1), f32),
            jax.ShapeDtypeStruct((L,), f32),
        ],
    )
    asv, adv, h0, h1, sh = prologue(
        x, W, att_src.reshape(1, C_OUT), att_dst.reshape(1, C_OUT))

    mesh = plsc.VectorSubcoreMesh(
        core_axis_name="c", subcore_axis_name="s", num_cores=NC, num_subcores=NS)
    sc = pl.kernel(
        _sc_body,
        out_type=(
            jax.ShapeDtypeStruct((NW, N), f32),
            jax.ShapeDtypeStruct((NW, N), f32),
            jax.ShapeDtypeStruct((NW, N), f32),
        ),
        mesh=mesh,
        compiler_params=pltpu.CompilerParams(
            needs_layout_passes=False, use_tc_tiling_on_sc=False),
        scratch_types=[
            pltpu.VMEM((N,), f32),
            pltpu.VMEM((N,), f32),
            pltpu.VMEM((N,), f32),
            pltpu.VMEM((N,), f32),
            pltpu.VMEM((L,), f32),
            pltpu.VMEM((EPW,), jnp.int32),
            pltpu.VMEM((EPW,), jnp.int32),
            pltpu.VMEM((N,), f32),
            pltpu.VMEM((N,), f32),
            pltpu.VMEM((N,), f32),
            pltpu.SemaphoreType.DMA,
        ],
    )
    dp, n0p, n1p = sc(asv.reshape(N), adv.reshape(N), h0.reshape(N),
                      h1.reshape(N), sh, edge_index)

    epilogue = pl.pallas_call(
        _epilogue_body,
        out_shape=jax.ShapeDtypeStruct((2, N), f32),
    )
    out2 = epilogue(dp, n0p, n1p, bias.reshape(C_OUT, 1))
    return out2.T


# speedup vs baseline: 165.6702x; 1.6744x over previous
import jax
import jax.numpy as jnp
from jax import lax
from jax.experimental import pallas as pl
from jax.experimental.pallas import tpu as pltpu
from jax.experimental.pallas import tpu_sc as plsc

N = 10000
E = 320000
D_IN = 192
C_OUT = 2

NC = 2
NS = 16
L = 16
NW = NC * NS

U = 5
EPW = E // NW
NIT = EPW // (U * L)
NSL = ((N + NW - 1) // NW + L - 1) // L * L
NP2 = 10240
CHK = NP2 // NS
ZIT = NP2 // L


def _prologue_body(xt_ref, w_ref, as_ref, ad_ref,
                   asv_ref, adv_ref, h0_ref, h1_ref, sh_ref):
    ht = lax.dot_general(w_ref[...], xt_ref[...],
                         dimension_numbers=(((0,), (0,)), ((), ())),
                         preferred_element_type=jnp.float32)
    asv = jnp.sum(ht * as_ref[...], axis=0, keepdims=True)
    adv = jnp.sum(ht * ad_ref[...], axis=0, keepdims=True)
    asv_ref[...] = asv[0]
    adv_ref[...] = adv[0]
    h0_ref[...] = ht[0]
    h1_ref[...] = ht[1]
    m = jnp.max(asv) + jnp.max(adv)
    sh = jnp.where(m >= 0.0, m, 0.2 * m)
    sh_ref[...] = jnp.full((L,), sh, jnp.float32)


def _sc_body(as_hbm, ad_hbm, h0_hbm, h1_hbm, sh_hbm, ei_hbm, i3_hbm,
             parts_hbm,
             as_v, ad_v, h0_v, h1_v, sh_v, src_v, dst_v,
             acc_v, i3_v, sh3, sem):
    cid = lax.axis_index("c")
    sid = lax.axis_index("s")
    wid = sid * NC + cid
    base = pl.multiple_of(wid * EPW, 8)
    copies = [
        pltpu.async_copy(as_hbm, as_v, sem),
        pltpu.async_copy(ad_hbm, ad_v, sem),
        pltpu.async_copy(h0_hbm, h0_v, sem),
        pltpu.async_copy(h1_hbm, h1_v, sem),
        pltpu.async_copy(sh_hbm, sh_v, sem),
        pltpu.async_copy(i3_hbm, i3_v, sem),
        pltpu.async_copy(ei_hbm.at[0, pl.ds(base, EPW)], src_v, sem),
        pltpu.async_copy(ei_hbm.at[1, pl.ds(base, EPW)], dst_v, sem),
    ]

    z = jnp.zeros((L,), jnp.float32)

    def zero_step(i, carry):
        off = pl.multiple_of(i * L, 8)
        acc_v[0, pl.ds(off, L)] = z
        acc_v[1, pl.ds(off, L)] = z
        acc_v[2, pl.ds(off, L)] = z
        return carry

    lax.fori_loop(0, ZIT, zero_step, 0)

    @pl.when(sid == 0)
    def _():
        pltpu.sync_copy(acc_v, sh3)

    for c in copies:
        c.wait()

    shift = sh_v[...]
    r0 = jnp.zeros((L,), jnp.int32)
    r1 = r0 + 1
    r2 = r0 + 2

    def step(i, carry):
        block = pl.multiple_of(i * (U * L), 8)
        for u in range(U):
            off = block + u * L
            s = src_v[pl.ds(off, L)]
            d = dst_v[pl.ds(off, L)]
            av = plsc.load_gather(as_v, [s]) + plsc.load_gather(ad_v, [d])
            av = jnp.maximum(av, 0.2 * av) - shift
            ex = jnp.exp(av)
            h0 = plsc.load_gather(h0_v, [s])
            h1 = plsc.load_gather(h1_v, [s])
            plsc.addupdate_scatter(acc_v, [r0, d], ex)
            plsc.addupdate_scatter(acc_v, [r1, d], ex * h0)
            plsc.addupdate_scatter(acc_v, [r2, d], ex * h1)
        return carry

    lax.fori_loop(0, NIT, step, 0)

    nbase = pl.multiple_of(wid * NSL, 8)
    nits = (jnp.minimum(NSL, N - nbase) + L - 1) // L

    def loop_step(j, carry):
        sl = pl.ds(nbase + j * L, L)
        av = as_v[sl] + ad_v[sl]
        av = jnp.maximum(av, 0.2 * av) - shift
        ex = jnp.exp(av)
        plsc.addupdate(acc_v.at[0, sl], ex)
        plsc.addupdate(acc_v.at[1, sl], ex * h0_v[sl])
        plsc.addupdate(acc_v.at[2, sl], ex * h1_v[sl])
        return carry

    lax.fori_loop(0, nits, loop_step, 0)

    plsc.subcore_barrier()
    pltpu.sync_copy(acc_v, sh3.at[i3_v], add=True)
    plsc.subcore_barrier()
    nb = pl.multiple_of(sid * CHK, 8)
    pltpu.sync_copy(sh3.at[:, pl.ds(nb, CHK)],
                    parts_hbm.at[cid, :, pl.ds(nb, CHK)])


def _epilogue_body(p_ref, b_ref, out_ref):
    q = p_ref[0] + p_ref[1]
    den = q[0:1, :N]
    n0 = q[1:2, :N]
    n1 = q[2:3, :N]
    inv = 1.0 / (den + 1e-16)
    out_ref[...] = jnp.concatenate([n0 * inv, n1 * inv], axis=0) + b_ref[...]


@jax.jit
def kernel(x, edge_index, edge_attr, W, att_src, att_dst, bias):
    del edge_attr
    f32 = jnp.float32

    prologue = pl.pallas_call(
        _prologue_body,
        out_shape=[
            jax.ShapeDtypeStruct((N,), f32),
            jax.ShapeDtypeStruct((N,), f32),
            jax.ShapeDtypeStruct((N,), f32),
            jax.ShapeDtypeStruct((N,), f32),
            jax.ShapeDtypeStruct((L,), f32),
        ],
    )
    asv, adv, h0, h1, sh = prologue(
        x.T, W, att_src.reshape(C_OUT, 1), att_dst.reshape(C_OUT, 1))

    i3 = jnp.arange(3, dtype=jnp.int32)

    mesh = plsc.VectorSubcoreMesh(
        core_axis_name="c", subcore_axis_name="s", num_cores=NC, num_subcores=NS)
    sc = pl.kernel(
        _sc_body,
        out_type=jax.ShapeDtypeStruct((NC, 3, NP2), f32),
        mesh=mesh,
        compiler_params=pltpu.CompilerParams(
            needs_layout_passes=False, use_tc_tiling_on_sc=False),
        scratch_types=[
            pltpu.VMEM((N,), f32),
            pltpu.VMEM((N,), f32),
            pltpu.VMEM((N,), f32),
            pltpu.VMEM((N,), f32),
            pltpu.VMEM((L,), f32),
            pltpu.VMEM((EPW,), jnp.int32),
            pltpu.VMEM((EPW,), jnp.int32),
            pltpu.VMEM((3, NP2), f32),
            pltpu.VMEM((3,), jnp.int32),
            pltpu.VMEM_SHARED((3, NP2), f32),
            pltpu.SemaphoreType.DMA,
        ],
    )
    parts = sc(asv, adv, h0, h1, sh, edge_index, i3)

    epilogue = pl.pallas_call(
        _epilogue_body,
        out_shape=jax.ShapeDtypeStruct((2, N), f32),
    )
    out2 = epilogue(parts, bias.reshape(C_OUT, 1))
    return out2.T
